# baseline (speedup 1.0000x reference)
import functools

import jax
import jax.numpy as jnp
from jax import lax
from jax.experimental import pallas as pl
from jax.experimental.pallas import tpu as pltpu
from jax.experimental.pallas import tpu_sc as plsc

NUM_EMB = 1000000
B = 16384
L = 50
D = 32
NC = 2
NS = 16
NW = NC * NS
BPW = B // NW
NBUF = 8

CCOL = 512
NUNIT = NUM_EMB // CCOL
NTAIL = NUM_EMB - NUNIT * CCOL


def _make_transpose_kernel():
    mesh = plsc.VectorSubcoreMesh(core_axis_name="c", subcore_axis_name="s")

    @functools.partial(
        pl.kernel,
        mesh=mesh,
        out_type=jax.ShapeDtypeStruct((NUM_EMB * D,), jnp.float32),
        compiler_params=pltpu.CompilerParams(needs_layout_passes=False),
        scratch_types=[
            pltpu.VMEM((D, CCOL), jnp.float32),
            pltpu.VMEM((D, CCOL), jnp.float32),
            pltpu.VMEM((CCOL * D,), jnp.float32),
            pltpu.VMEM((CCOL * D,), jnp.float32),
            pltpu.SemaphoreType.DMA((2,)),
            pltpu.SemaphoreType.DMA((2,)),
        ],
    )
    def k(src_hbm, tail_hbm, dst_hbm, vbuf0, vbuf1, dbuf0, dbuf1,
          sem_in, sem_out):
        vbufs = (vbuf0, vbuf1)
        dbufs = (dbuf0, dbuf1)
        wid = lax.axis_index("s") * NC + lax.axis_index("c")
        lo = wid * NUNIT // NW
        hi = (wid + 1) * NUNIT // NW
        nu = hi - lo
        lane16 = lax.iota(jnp.int32, 16) * D

        def start_in(u, b):
            pltpu.async_copy(
                src_hbm.at[:, pl.ds(u * CCOL, CCOL)], vbufs[b], sem_in.at[b]
            )

        start_in(lo, 0)

        def transpose_unit(b):
            def j16_body(j16, carry):
                dsub = dbufs[b].at[pl.ds(j16 * 16 * D, 16 * D)]
                for c in range(D):
                    x16 = vbufs[b][c, pl.ds(j16 * 16, 16)]
                    plsc.store_scatter(dsub, [lane16 + c], x16)
                return carry

            lax.fori_loop(0, CCOL // 16, j16_body, 0)

        def pair_body(p, carry):
            for b in range(2):
                g = p * 2 + b
                u = lo + g

                @pl.when(g < nu)
                def _():
                    @pl.when(g + 1 < nu)
                    def _():
                        start_in(u + 1, 1 - b)

                    pltpu.make_async_copy(
                        src_hbm.at[:, pl.ds(0, CCOL)], vbufs[b], sem_in.at[b]
                    ).wait()

                    @pl.when(g >= 2)
                    def _():
                        pltpu.make_async_copy(
                            dbufs[b], dst_hbm.at[pl.ds(0, CCOL * D)],
                            sem_out.at[b],
                        ).wait()

                    transpose_unit(b)
                    pltpu.async_copy(
                        dbufs[b], dst_hbm.at[pl.ds(u * CCOL * D, CCOL * D)],
                        sem_out.at[b],
                    )

            return carry

        lax.fori_loop(0, (nu + 1) // 2, pair_body, 0)
        for b in range(2):
            pltpu.make_async_copy(
                dbufs[b], dst_hbm.at[pl.ds(0, CCOL * D)], sem_out.at[b]
            ).wait()

        @pl.when(wid == NW - 1)
        def _():
            pltpu.sync_copy(tail_hbm, dbuf0.at[pl.ds(0, NTAIL * D)])
            pltpu.sync_copy(
                dbuf0.at[pl.ds(0, NTAIL * D)],
                dst_hbm.at[pl.ds(NUNIT * CCOL * D, NTAIL * D)],
            )

    return k


def _make_gather_kernel():
    mesh = plsc.VectorSubcoreMesh(core_axis_name="c", subcore_axis_name="s")

    @functools.partial(
        pl.kernel,
        mesh=mesh,
        out_type=jax.ShapeDtypeStruct((B, D), jnp.float32),
        compiler_params=pltpu.CompilerParams(use_tc_tiling_on_sc=False),
        scratch_types=[
            pltpu.VMEM((BPW, L), jnp.int32),
            pltpu.VMEM((NBUF, L, D), jnp.float32),
            pltpu.VMEM((BPW, D), jnp.float32),
            pltpu.SemaphoreType.DMA((NBUF,)),
        ],
    )
    def k(idx_hbm, table_hbm, out_hbm, idx_v, buf_v, out_v, sems):
        wid = lax.axis_index("s") * NC + lax.axis_index("c")
        base = wid * BPW
        pltpu.sync_copy(idx_hbm.at[pl.ds(base, BPW)], idx_v)

        for b in range(NBUF):
            pltpu.async_copy(table_hbm.at[idx_v.at[b]], buf_v.at[b], sems.at[b])

        def group_body(g, carry):
            for b in range(NBUF):
                s = g * NBUF + b
                pltpu.make_async_copy(
                    table_hbm.at[idx_v.at[0]], buf_v.at[b], sems.at[b]
                ).wait()
                a0 = buf_v[b, 0, pl.ds(0, 16)]
                a1 = buf_v[b, 0, pl.ds(16, 16)]
                for j in range(1, L):
                    a0 = a0 + buf_v[b, j, pl.ds(0, 16)]
                    a1 = a1 + buf_v[b, j, pl.ds(16, 16)]
                out_v[s, pl.ds(0, 16)] = a0
                out_v[s, pl.ds(16, 16)] = a1
                nxt = s + NBUF

                @pl.when(nxt < BPW)
                def _():
                    pltpu.async_copy(
                        table_hbm.at[idx_v.at[nxt]], buf_v.at[b], sems.at[b]
                    )

            return carry

        lax.fori_loop(0, BPW // NBUF, group_body, 0)
        pltpu.sync_copy(out_v, out_hbm.at[pl.ds(base, BPW)])

    return k


_transpose = _make_transpose_kernel()
_gather = _make_gather_kernel()


def kernel(x, table):
    tail = table[NUNIT * CCOL :].reshape(-1)
    lin = _transpose(table.T, tail)
    tbl = lin.reshape(NUM_EMB, D)
    idx = x.astype(jnp.int32)
    return _gather(idx, tbl)


# speedup vs baseline: 2.1301x; 2.1301x over previous
import functools

import jax
import jax.numpy as jnp
from jax import lax
from jax.experimental import pallas as pl
from jax.experimental.pallas import tpu as pltpu
from jax.experimental.pallas import tpu_sc as plsc

NUM_EMB = 1000000
B = 16384
L = 50
D = 32
NC = 2
NS = 16
NW = NC * NS
BPW = B // NW
NBUF = 8

CCOL = 512
NUNIT = NUM_EMB // CCOL
NTAIL = NUM_EMB - NUNIT * CCOL


def _make_transpose_kernel():
    mesh = plsc.VectorSubcoreMesh(core_axis_name="c", subcore_axis_name="s")

    @functools.partial(
        pl.kernel,
        mesh=mesh,
        out_type=jax.ShapeDtypeStruct((NUM_EMB * D,), jnp.float32),
        compiler_params=pltpu.CompilerParams(needs_layout_passes=False),
        scratch_types=[
            pltpu.VMEM((D, CCOL), jnp.float32),
            pltpu.VMEM((D, CCOL), jnp.float32),
            pltpu.VMEM((CCOL * D,), jnp.float32),
            pltpu.VMEM((CCOL * D,), jnp.float32),
            pltpu.SemaphoreType.DMA((2,)),
            pltpu.SemaphoreType.DMA((2,)),
        ],
    )
    def k(src_hbm, tail_hbm, dst_hbm, vbuf0, vbuf1, dbuf0, dbuf1,
          sem_in, sem_out):
        vbufs = (vbuf0, vbuf1)
        dbufs = (dbuf0, dbuf1)
        wid = lax.axis_index("s") * NC + lax.axis_index("c")
        lo = wid * NUNIT // NW
        hi = (wid + 1) * NUNIT // NW
        nu = hi - lo
        lane16 = lax.iota(jnp.int32, 16) * D

        def start_in(u, b):
            pltpu.async_copy(
                src_hbm.at[:, pl.ds(u * CCOL, CCOL)], vbufs[b], sem_in.at[b]
            )

        start_in(lo, 0)

        iota = lax.iota(jnp.int32, 16)

        def transpose_unit(b):
            def j16_body(j16, carry):
                colidx = j16 * 16 + iota
                dsub = dbufs[b].at[pl.ds(j16 * 16 * D, 16 * D)]
                for chalf in range(2):
                    for k in range(16):
                        diag = (k + iota) & 15
                        rowidx = chalf * 16 + diag
                        x16 = plsc.load_gather(vbufs[b], [rowidx, colidx])
                        plsc.store_scatter(
                            dsub, [iota * D + chalf * 16 + diag], x16
                        )
                return carry

            lax.fori_loop(0, CCOL // 16, j16_body, 0)

        def pair_body(p, carry):
            for b in range(2):
                g = p * 2 + b
                u = lo + g

                @pl.when(g < nu)
                def _():
                    @pl.when(g + 1 < nu)
                    def _():
                        start_in(u + 1, 1 - b)

                    pltpu.make_async_copy(
                        src_hbm.at[:, pl.ds(0, CCOL)], vbufs[b], sem_in.at[b]
                    ).wait()

                    @pl.when(g >= 2)
                    def _():
                        pltpu.make_async_copy(
                            dbufs[b], dst_hbm.at[pl.ds(0, CCOL * D)],
                            sem_out.at[b],
                        ).wait()

                    transpose_unit(b)
                    pltpu.async_copy(
                        dbufs[b], dst_hbm.at[pl.ds(u * CCOL * D, CCOL * D)],
                        sem_out.at[b],
                    )

            return carry

        lax.fori_loop(0, (nu + 1) // 2, pair_body, 0)
        for b in range(2):
            pltpu.make_async_copy(
                dbufs[b], dst_hbm.at[pl.ds(0, CCOL * D)], sem_out.at[b]
            ).wait()

        @pl.when(wid == NW - 1)
        def _():
            pltpu.sync_copy(tail_hbm, dbuf0.at[pl.ds(0, NTAIL * D)])
            pltpu.sync_copy(
                dbuf0.at[pl.ds(0, NTAIL * D)],
                dst_hbm.at[pl.ds(NUNIT * CCOL * D, NTAIL * D)],
            )

    return k


def _make_gather_kernel():
    mesh = plsc.VectorSubcoreMesh(core_axis_name="c", subcore_axis_name="s")

    @functools.partial(
        pl.kernel,
        mesh=mesh,
        out_type=jax.ShapeDtypeStruct((B, D), jnp.float32),
        compiler_params=pltpu.CompilerParams(use_tc_tiling_on_sc=False),
        scratch_types=[
            pltpu.VMEM((BPW, L), jnp.int32),
            pltpu.VMEM((NBUF, L, D), jnp.float32),
            pltpu.VMEM((BPW, D), jnp.float32),
            pltpu.SemaphoreType.DMA((NBUF,)),
        ],
    )
    def k(idx_hbm, table_hbm, out_hbm, idx_v, buf_v, out_v, sems):
        wid = lax.axis_index("s") * NC + lax.axis_index("c")
        base = wid * BPW
        pltpu.sync_copy(idx_hbm.at[pl.ds(base, BPW)], idx_v)

        for b in range(NBUF):
            pltpu.async_copy(table_hbm.at[idx_v.at[b]], buf_v.at[b], sems.at[b])

        def group_body(g, carry):
            for b in range(NBUF):
                s = g * NBUF + b
                pltpu.make_async_copy(
                    table_hbm.at[idx_v.at[0]], buf_v.at[b], sems.at[b]
                ).wait()
                a0 = buf_v[b, 0, pl.ds(0, 16)]
                a1 = buf_v[b, 0, pl.ds(16, 16)]
                for j in range(1, L):
                    a0 = a0 + buf_v[b, j, pl.ds(0, 16)]
                    a1 = a1 + buf_v[b, j, pl.ds(16, 16)]
                out_v[s, pl.ds(0, 16)] = a0
                out_v[s, pl.ds(16, 16)] = a1
                nxt = s + NBUF

                @pl.when(nxt < BPW)
                def _():
                    pltpu.async_copy(
                        table_hbm.at[idx_v.at[nxt]], buf_v.at[b], sems.at[b]
                    )

            return carry

        lax.fori_loop(0, BPW // NBUF, group_body, 0)
        pltpu.sync_copy(out_v, out_hbm.at[pl.ds(base, BPW)])

    return k


_transpose = _make_transpose_kernel()
_gather = _make_gather_kernel()


def kernel(x, table):
    tail = table[NUNIT * CCOL :].reshape(-1)
    lin = _transpose(table.T, tail)
    tbl = lin.reshape(NUM_EMB, D)
    idx = x.astype(jnp.int32)
    return _gather(idx, tbl)


# speedup vs baseline: 2.4358x; 1.1435x over previous
import functools

import jax
import jax.numpy as jnp
from jax import lax
from jax.experimental import pallas as pl
from jax.experimental.pallas import tpu as pltpu
from jax.experimental.pallas import tpu_sc as plsc

NUM_EMB = 1000000
B = 16384
L = 50
D = 32
NC = 2
NS = 16
NW = NC * NS
BPW = B // NW
NBUF = 8

CCOL = 512
NUNIT = NUM_EMB // CCOL
NTAIL = NUM_EMB - NUNIT * CCOL


def _make_transpose_kernel():
    mesh = plsc.VectorSubcoreMesh(core_axis_name="c", subcore_axis_name="s")

    @functools.partial(
        pl.kernel,
        mesh=mesh,
        out_type=jax.ShapeDtypeStruct((NUM_EMB * D,), jnp.float32),
        compiler_params=pltpu.CompilerParams(needs_layout_passes=False),
        scratch_types=[
            pltpu.VMEM((D, CCOL), jnp.float32),
            pltpu.VMEM((D, CCOL), jnp.float32),
            pltpu.VMEM((CCOL * D,), jnp.float32),
            pltpu.VMEM((CCOL * D,), jnp.float32),
            pltpu.SemaphoreType.DMA((2,)),
            pltpu.SemaphoreType.DMA((2,)),
        ],
    )
    def k(src_hbm, tail_hbm, dst_hbm, vbuf0, vbuf1, dbuf0, dbuf1,
          sem_in, sem_out):
        vbufs = (vbuf0, vbuf1)
        dbufs = (dbuf0, dbuf1)
        wid = lax.axis_index("s") * NC + lax.axis_index("c")
        lo = wid * NUNIT // NW
        hi = (wid + 1) * NUNIT // NW
        nu = hi - lo
        lane16 = lax.iota(jnp.int32, 16) * D

        def start_in(u, b):
            pltpu.async_copy(
                src_hbm.at[:, pl.ds(u * CCOL, CCOL)], vbufs[b], sem_in.at[b]
            )

        start_in(lo, 0)

        iota = lax.iota(jnp.int32, 16)

        def transpose_unit(b):
            def j16_body(j16, carry):
                colidx = j16 * 16 + iota
                dsub = dbufs[b].at[pl.ds(j16 * 16 * D, 16 * D)]
                for chalf in range(2):
                    xs = []
                    for k in range(16):
                        diag = (k + iota) & 15
                        xs.append(
                            plsc.load_gather(vbufs[b], [chalf * 16 + diag, colidx])
                        )
                    for k in range(16):
                        diag = (k + iota) & 15
                        plsc.store_scatter(
                            dsub, [iota * D + chalf * 16 + diag], xs[k]
                        )
                return carry

            lax.fori_loop(0, CCOL // 16, j16_body, 0)

        def pair_body(p, carry):
            for b in range(2):
                g = p * 2 + b
                u = lo + g

                @pl.when(g < nu)
                def _():
                    @pl.when(g + 1 < nu)
                    def _():
                        start_in(u + 1, 1 - b)

                    pltpu.make_async_copy(
                        src_hbm.at[:, pl.ds(0, CCOL)], vbufs[b], sem_in.at[b]
                    ).wait()

                    @pl.when(g >= 2)
                    def _():
                        pltpu.make_async_copy(
                            dbufs[b], dst_hbm.at[pl.ds(0, CCOL * D)],
                            sem_out.at[b],
                        ).wait()

                    transpose_unit(b)
                    pltpu.async_copy(
                        dbufs[b], dst_hbm.at[pl.ds(u * CCOL * D, CCOL * D)],
                        sem_out.at[b],
                    )

            return carry

        lax.fori_loop(0, (nu + 1) // 2, pair_body, 0)
        for b in range(2):
            pltpu.make_async_copy(
                dbufs[b], dst_hbm.at[pl.ds(0, CCOL * D)], sem_out.at[b]
            ).wait()

        @pl.when(wid == NW - 1)
        def _():
            pltpu.sync_copy(tail_hbm, dbuf0.at[pl.ds(0, NTAIL * D)])
            pltpu.sync_copy(
                dbuf0.at[pl.ds(0, NTAIL * D)],
                dst_hbm.at[pl.ds(NUNIT * CCOL * D, NTAIL * D)],
            )

    return k


def _make_gather_kernel():
    mesh = plsc.VectorSubcoreMesh(core_axis_name="c", subcore_axis_name="s")

    @functools.partial(
        pl.kernel,
        mesh=mesh,
        out_type=jax.ShapeDtypeStruct((B, D), jnp.float32),
        compiler_params=pltpu.CompilerParams(use_tc_tiling_on_sc=False),
        scratch_types=[
            pltpu.VMEM((BPW, L), jnp.int32),
            pltpu.VMEM((NBUF, L, D), jnp.float32),
            pltpu.VMEM((BPW, D), jnp.float32),
            pltpu.SemaphoreType.DMA((NBUF,)),
        ],
    )
    def k(idx_hbm, table_hbm, out_hbm, idx_v, buf_v, out_v, sems):
        wid = lax.axis_index("s") * NC + lax.axis_index("c")
        base = wid * BPW
        pltpu.sync_copy(idx_hbm.at[pl.ds(base, BPW)], idx_v)

        for b in range(NBUF):
            pltpu.async_copy(table_hbm.at[idx_v.at[b]], buf_v.at[b], sems.at[b])

        def group_body(g, carry):
            for b in range(NBUF):
                s = g * NBUF + b
                pltpu.make_async_copy(
                    table_hbm.at[idx_v.at[0]], buf_v.at[b], sems.at[b]
                ).wait()
                a0 = buf_v[b, 0, pl.ds(0, 16)]
                a1 = buf_v[b, 0, pl.ds(16, 16)]
                for j in range(1, L):
                    a0 = a0 + buf_v[b, j, pl.ds(0, 16)]
                    a1 = a1 + buf_v[b, j, pl.ds(16, 16)]
                out_v[s, pl.ds(0, 16)] = a0
                out_v[s, pl.ds(16, 16)] = a1
                nxt = s + NBUF

                @pl.when(nxt < BPW)
                def _():
                    pltpu.async_copy(
                        table_hbm.at[idx_v.at[nxt]], buf_v.at[b], sems.at[b]
                    )

            return carry

        lax.fori_loop(0, BPW // NBUF, group_body, 0)
        pltpu.sync_copy(out_v, out_hbm.at[pl.ds(base, BPW)])

    return k


_transpose = _make_transpose_kernel()
_gather = _make_gather_kernel()


def kernel(x, table):
    tail = table[NUNIT * CCOL :].reshape(-1)
    lin = _transpose(table.T, tail)
    tbl = lin.reshape(NUM_EMB, D)
    idx = x.astype(jnp.int32)
    return _gather(idx, tbl)


# speedup vs baseline: 2.9720x; 1.2201x over previous
import functools

import jax
import jax.numpy as jnp
from jax import lax
from jax.experimental import pallas as pl
from jax.experimental.pallas import tpu as pltpu
from jax.experimental.pallas import tpu_sc as plsc

NUM_EMB = 1000000
B = 16384
L = 50
D = 32
NC = 2
NS = 16
NW = NC * NS
BPW = B // NW
NBUF = 8

CCOL = 512
NUNIT = NUM_EMB // CCOL
NTAIL = NUM_EMB - NUNIT * CCOL


def _make_transpose_kernel():
    mesh = plsc.VectorSubcoreMesh(core_axis_name="c", subcore_axis_name="s")

    @functools.partial(
        pl.kernel,
        mesh=mesh,
        out_type=jax.ShapeDtypeStruct((NUM_EMB * D,), jnp.float32),
        compiler_params=pltpu.CompilerParams(needs_layout_passes=False),
        scratch_types=[
            pltpu.VMEM((D, CCOL), jnp.float32),
            pltpu.VMEM((D, CCOL), jnp.float32),
            pltpu.VMEM((CCOL * D,), jnp.float32),
            pltpu.VMEM((CCOL * D,), jnp.float32),
            pltpu.SemaphoreType.DMA((2,)),
            pltpu.SemaphoreType.DMA((2,)),
        ],
    )
    def k(src_hbm, tail_hbm, dst_hbm, vbuf0, vbuf1, dbuf0, dbuf1,
          sem_in, sem_out):
        vbufs = (vbuf0, vbuf1)
        dbufs = (dbuf0, dbuf1)
        wid = lax.axis_index("s") * NC + lax.axis_index("c")
        lo = wid * NUNIT // NW
        hi = (wid + 1) * NUNIT // NW
        nu = hi - lo
        lane16 = lax.iota(jnp.int32, 16) * D

        def start_in(u, b):
            pltpu.async_copy(
                src_hbm.at[:, pl.ds(u * CCOL, CCOL)], vbufs[b], sem_in.at[b]
            )

        start_in(lo, 0)

        iota = lax.iota(jnp.int32, 16)

        def transpose_unit(b):
            def j16_body(j16h, carry):
                for half in range(2):
                    j16 = j16h * 2 + half
                    colidx = j16 * 16 + iota
                    dsub = dbufs[b].at[pl.ds(j16 * 16 * D, 16 * D)]
                    for chalf in range(2):
                        xs = []
                        for k in range(16):
                            diag = (k + iota) & 15
                            xs.append(
                                plsc.load_gather(
                                    vbufs[b], [chalf * 16 + diag, colidx]
                                )
                            )
                        for k in range(16):
                            diag = (k + iota) & 15
                            plsc.store_scatter(
                                dsub, [iota * D + chalf * 16 + diag], xs[k]
                            )
                return carry

            lax.fori_loop(0, CCOL // 32, j16_body, 0)

        def pair_body(p, carry):
            for b in range(2):
                g = p * 2 + b
                u = lo + g

                @pl.when(g < nu)
                def _():
                    @pl.when(g + 1 < nu)
                    def _():
                        start_in(u + 1, 1 - b)

                    pltpu.make_async_copy(
                        src_hbm.at[:, pl.ds(0, CCOL)], vbufs[b], sem_in.at[b]
                    ).wait()

                    @pl.when(g >= 2)
                    def _():
                        pltpu.make_async_copy(
                            dbufs[b], dst_hbm.at[pl.ds(0, CCOL * D)],
                            sem_out.at[b],
                        ).wait()

                    transpose_unit(b)
                    pltpu.async_copy(
                        dbufs[b], dst_hbm.at[pl.ds(u * CCOL * D, CCOL * D)],
                        sem_out.at[b],
                    )

            return carry

        lax.fori_loop(0, (nu + 1) // 2, pair_body, 0)
        for b in range(2):
            pltpu.make_async_copy(
                dbufs[b], dst_hbm.at[pl.ds(0, CCOL * D)], sem_out.at[b]
            ).wait()

        @pl.when(wid == NW - 1)
        def _():
            pltpu.sync_copy(tail_hbm, dbuf0.at[pl.ds(0, NTAIL * D)])
            pltpu.sync_copy(
                dbuf0.at[pl.ds(0, NTAIL * D)],
                dst_hbm.at[pl.ds(NUNIT * CCOL * D, NTAIL * D)],
            )

    return k


def _make_gather_kernel():
    mesh = plsc.VectorSubcoreMesh(core_axis_name="c", subcore_axis_name="s")

    @functools.partial(
        pl.kernel,
        mesh=mesh,
        out_type=jax.ShapeDtypeStruct((B, D), jnp.float32),
        compiler_params=pltpu.CompilerParams(use_tc_tiling_on_sc=False),
        scratch_types=[
            pltpu.VMEM((BPW, L), jnp.int32),
            pltpu.VMEM((NBUF, L, D), jnp.float32),
            pltpu.VMEM((BPW, D), jnp.float32),
            pltpu.SemaphoreType.DMA((NBUF,)),
        ],
    )
    def k(idx_hbm, table_hbm, out_hbm, idx_v, buf_v, out_v, sems):
        wid = lax.axis_index("s") * NC + lax.axis_index("c")
        base = wid * BPW
        pltpu.sync_copy(idx_hbm.at[pl.ds(base, BPW)], idx_v)

        for b in range(NBUF):
            pltpu.async_copy(table_hbm.at[idx_v.at[b]], buf_v.at[b], sems.at[b])

        def group_body(g, carry):
            for b in range(NBUF):
                s = g * NBUF + b
                pltpu.make_async_copy(
                    table_hbm.at[idx_v.at[0]], buf_v.at[b], sems.at[b]
                ).wait()
                a0 = buf_v[b, 0, pl.ds(0, 16)]
                a1 = buf_v[b, 0, pl.ds(16, 16)]
                for j in range(1, L):
                    a0 = a0 + buf_v[b, j, pl.ds(0, 16)]
                    a1 = a1 + buf_v[b, j, pl.ds(16, 16)]
                out_v[s, pl.ds(0, 16)] = a0
                out_v[s, pl.ds(16, 16)] = a1
                nxt = s + NBUF

                @pl.when(nxt < BPW)
                def _():
                    pltpu.async_copy(
                        table_hbm.at[idx_v.at[nxt]], buf_v.at[b], sems.at[b]
                    )

            return carry

        lax.fori_loop(0, BPW // NBUF, group_body, 0)
        pltpu.sync_copy(out_v, out_hbm.at[pl.ds(base, BPW)])

    return k


_transpose = _make_transpose_kernel()
_gather = _make_gather_kernel()


def kernel(x, table):
    tail = table[NUNIT * CCOL :].reshape(-1)
    lin = _transpose(table.T, tail)
    tbl = lin.reshape(NUM_EMB, D)
    idx = x.astype(jnp.int32)
    return _gather(idx, tbl)


# speedup vs baseline: 3.3540x; 1.1285x over previous
import functools

import jax
import jax.numpy as jnp
from jax import lax
from jax.experimental import pallas as pl
from jax.experimental.pallas import tpu as pltpu
from jax.experimental.pallas import tpu_sc as plsc

NUM_EMB = 1000000
B = 16384
L = 50
D = 32
NC = 2
NS = 16
NW = NC * NS
BPW = B // NW
NBUF = 8

CCOL = 512
NUNIT = NUM_EMB // CCOL
NTAIL = NUM_EMB - NUNIT * CCOL


def _make_transpose_kernel():
    mesh = plsc.VectorSubcoreMesh(core_axis_name="c", subcore_axis_name="s")

    @functools.partial(
        pl.kernel,
        mesh=mesh,
        out_type=jax.ShapeDtypeStruct((NUM_EMB * D,), jnp.float32),
        compiler_params=pltpu.CompilerParams(needs_layout_passes=False),
        scratch_types=[
            pltpu.VMEM((D, CCOL), jnp.float32),
            pltpu.VMEM((D, CCOL), jnp.float32),
            pltpu.VMEM((CCOL * D,), jnp.float32),
            pltpu.VMEM((CCOL * D,), jnp.float32),
            pltpu.SemaphoreType.DMA((2,)),
            pltpu.SemaphoreType.DMA((2,)),
        ],
    )
    def k(src_hbm, tail_hbm, dst_hbm, vbuf0, vbuf1, dbuf0, dbuf1,
          sem_in, sem_out):
        vbufs = (vbuf0, vbuf1)
        dbufs = (dbuf0, dbuf1)
        wid = lax.axis_index("s") * NC + lax.axis_index("c")
        lo = wid * NUNIT // NW
        hi = (wid + 1) * NUNIT // NW
        nu = hi - lo
        lane16 = lax.iota(jnp.int32, 16) * D

        def start_in(u, b):
            pltpu.async_copy(
                src_hbm.at[:, pl.ds(u * CCOL, CCOL)], vbufs[b], sem_in.at[b]
            )

        start_in(lo, 0)

        iota = lax.iota(jnp.int32, 16)

        def transpose_unit(b):
            def j16_body(j16h, carry):
                for half in range(4):
                    j16 = j16h * 4 + half
                    colidx = j16 * 16 + iota
                    dsub = dbufs[b].at[pl.ds(j16 * 16 * D, 16 * D)]
                    for chalf in range(2):
                        xs = []
                        for k in range(16):
                            diag = (k + iota) & 15
                            xs.append(
                                plsc.load_gather(
                                    vbufs[b], [chalf * 16 + diag, colidx]
                                )
                            )
                        for k in range(16):
                            diag = (k + iota) & 15
                            plsc.store_scatter(
                                dsub, [iota * D + chalf * 16 + diag], xs[k]
                            )
                return carry

            lax.fori_loop(0, CCOL // 64, j16_body, 0)

        def pair_body(p, carry):
            for b in range(2):
                g = p * 2 + b
                u = lo + g

                @pl.when(g < nu)
                def _():
                    @pl.when(g + 1 < nu)
                    def _():
                        start_in(u + 1, 1 - b)

                    pltpu.make_async_copy(
                        src_hbm.at[:, pl.ds(0, CCOL)], vbufs[b], sem_in.at[b]
                    ).wait()

                    @pl.when(g >= 2)
                    def _():
                        pltpu.make_async_copy(
                            dbufs[b], dst_hbm.at[pl.ds(0, CCOL * D)],
                            sem_out.at[b],
                        ).wait()

                    transpose_unit(b)
                    pltpu.async_copy(
                        dbufs[b], dst_hbm.at[pl.ds(u * CCOL * D, CCOL * D)],
                        sem_out.at[b],
                    )

            return carry

        lax.fori_loop(0, (nu + 1) // 2, pair_body, 0)
        for b in range(2):
            pltpu.make_async_copy(
                dbufs[b], dst_hbm.at[pl.ds(0, CCOL * D)], sem_out.at[b]
            ).wait()

        @pl.when(wid == NW - 1)
        def _():
            pltpu.sync_copy(tail_hbm, dbuf0.at[pl.ds(0, NTAIL * D)])
            pltpu.sync_copy(
                dbuf0.at[pl.ds(0, NTAIL * D)],
                dst_hbm.at[pl.ds(NUNIT * CCOL * D, NTAIL * D)],
            )

    return k


def _make_gather_kernel():
    mesh = plsc.VectorSubcoreMesh(core_axis_name="c", subcore_axis_name="s")

    @functools.partial(
        pl.kernel,
        mesh=mesh,
        out_type=jax.ShapeDtypeStruct((B, D), jnp.float32),
        compiler_params=pltpu.CompilerParams(use_tc_tiling_on_sc=False),
        scratch_types=[
            pltpu.VMEM((BPW, L), jnp.int32),
            pltpu.VMEM((NBUF, L, D), jnp.float32),
            pltpu.VMEM((BPW, D), jnp.float32),
            pltpu.SemaphoreType.DMA((NBUF,)),
        ],
    )
    def k(idx_hbm, table_hbm, out_hbm, idx_v, buf_v, out_v, sems):
        wid = lax.axis_index("s") * NC + lax.axis_index("c")
        base = wid * BPW
        pltpu.sync_copy(idx_hbm.at[pl.ds(base, BPW)], idx_v)

        for b in range(NBUF):
            pltpu.async_copy(table_hbm.at[idx_v.at[b]], buf_v.at[b], sems.at[b])

        def group_body(g, carry):
            for b in range(NBUF):
                s = g * NBUF + b
                pltpu.make_async_copy(
                    table_hbm.at[idx_v.at[0]], buf_v.at[b], sems.at[b]
                ).wait()
                a0 = buf_v[b, 0, pl.ds(0, 16)]
                a1 = buf_v[b, 0, pl.ds(16, 16)]
                for j in range(1, L):
                    a0 = a0 + buf_v[b, j, pl.ds(0, 16)]
                    a1 = a1 + buf_v[b, j, pl.ds(16, 16)]
                out_v[s, pl.ds(0, 16)] = a0
                out_v[s, pl.ds(16, 16)] = a1
                nxt = s + NBUF

                @pl.when(nxt < BPW)
                def _():
                    pltpu.async_copy(
                        table_hbm.at[idx_v.at[nxt]], buf_v.at[b], sems.at[b]
                    )

            return carry

        lax.fori_loop(0, BPW // NBUF, group_body, 0)
        pltpu.sync_copy(out_v, out_hbm.at[pl.ds(base, BPW)])

    return k


_transpose = _make_transpose_kernel()
_gather = _make_gather_kernel()


def kernel(x, table):
    tail = table[NUNIT * CCOL :].reshape(-1)
    lin = _transpose(table.T, tail)
    tbl = lin.reshape(NUM_EMB, D)
    idx = x.astype(jnp.int32)
    return _gather(idx, tbl)
